# baseline (speedup 1.0000x reference)
import functools

import jax
import jax.numpy as jnp
from jax import lax
from jax.experimental import pallas as pl
from jax.experimental.pallas import tpu as pltpu
from jax.experimental.pallas import tpu_sc as plsc

_B = 16384
_D = 64
_NBR = 5
_K = _NBR + 1
_NC, _NS, _L = 2, 16, 16
_NW = _NC * _NS
_BPW = _B // _NW
_C = 16
_NCH = _BPW // _C
_DJ = _D // _L
_AUX = 32


def _sc_body(df_t, pos_t, neg_t, it_t,
             idx_df_h, idx_pos_h, idx_neg_h, idx_it_h, aux_h,
             out_h,
             idxdf_v, idxpos_v, idxneg_v, idxit_v, aux_v,
             rdf_v, rpos_v, rneg_v, rit_v, obuf_v,
             sem_df, sem_pos, sem_neg, sem_it):
    wid = lax.axis_index("s") * _NC + lax.axis_index("c")
    base0 = pl.multiple_of(wid * _BPW, _BPW)

    pltpu.sync_copy(idx_df_h.at[pl.ds(base0 * _K, _BPW * _K)], idxdf_v)
    pltpu.sync_copy(idx_pos_h.at[pl.ds(base0 * _K, _BPW * _K)], idxpos_v)
    pltpu.sync_copy(idx_neg_h.at[pl.ds(base0 * _K, _BPW * _K)], idxneg_v)
    pltpu.sync_copy(idx_it_h.at[pl.ds(base0, _BPW)], idxit_v)
    pltpu.sync_copy(aux_h.at[pl.ds(base0 * _AUX, _BPW * _AUX)], aux_v)

    def half(rv, row, h, j):
        lo = rv[row, pl.ds(j * _L, _L)]
        hi = rv[row, pl.ds(_D + j * _L, _L)]
        return lo + h * (hi - lo)

    def chunk(c, carry):
        base = base0 + c * _C
        cp_df = pltpu.async_copy(
            df_t.at[idxdf_v.at[pl.ds(c * _C * _K, _C * _K)]], rdf_v, sem_df)
        cp_pos = pltpu.async_copy(
            pos_t.at[idxpos_v.at[pl.ds(c * _C * _K, _C * _K)]], rpos_v, sem_pos)
        cp_neg = pltpu.async_copy(
            neg_t.at[idxneg_v.at[pl.ds(c * _C * _K, _C * _K)]], rneg_v, sem_neg)
        cp_it = pltpu.async_copy(
            it_t.at[idxit_v.at[pl.ds(c * _C, _C)]], rit_v, sem_it)
        cp_df.wait()
        cp_pos.wait()
        cp_neg.wait()
        cp_it.wait()

        def row(b, carry2):
            r0 = b * _K
            ao = (c * _C + b) * _AUX
            a0 = aux_v[pl.ds(ao, _L)]
            a1 = aux_v[pl.ds(ao + _L, _L)]
            for j in range(_DJ):
                e = half(rdf_v, r0, a0[10], j)
                s = e
                q = e * e
                for n in range(1, _NBR):
                    e = half(rdf_v, r0 + n, a0[10 + n], j)
                    s = s + e
                    q = q + e * e
                tgt = half(rdf_v, r0 + _NBR, a0[15], j)
                obuf_v[b, pl.ds(j * _L, _L)] = s * s - q + tgt
                accp = half(rpos_v, r0 + _NBR, a1[5], j)
                accn = half(rneg_v, r0 + _NBR, a1[11], j)
                for n in range(_NBR):
                    accp = accp + a0[n] * half(rpos_v, r0 + n, a1[n], j)
                    accn = accn + a0[5 + n] * half(rneg_v, r0 + n, a1[6 + n], j)
                obuf_v[b, pl.ds(_D + j * _L, _L)] = accp
                obuf_v[b, pl.ds(2 * _D + j * _L, _L)] = accn
                obuf_v[b, pl.ds(3 * _D + j * _L, _L)] = half(rit_v, b, a1[12], j)
            return carry2

        lax.fori_loop(0, _C, row, 0)
        pltpu.sync_copy(obuf_v, out_h.at[pl.ds(base, _C)])
        return carry

    lax.fori_loop(0, _NCH, chunk, 0)


_sc_gather = functools.partial(
    pl.kernel,
    out_type=jax.ShapeDtypeStruct((_B, 4 * _D), jnp.float32),
    mesh=plsc.VectorSubcoreMesh(core_axis_name="c", subcore_axis_name="s",
                                num_cores=_NC, num_subcores=_NS),
    scratch_types=[
        pltpu.VMEM((_BPW * _K,), jnp.int32),
        pltpu.VMEM((_BPW * _K,), jnp.int32),
        pltpu.VMEM((_BPW * _K,), jnp.int32),
        pltpu.VMEM((_BPW,), jnp.int32),
        pltpu.VMEM((_BPW * _AUX,), jnp.float32),
        pltpu.VMEM((_C * _K, 2 * _D), jnp.float32),
        pltpu.VMEM((_C * _K, 2 * _D), jnp.float32),
        pltpu.VMEM((_C * _K, 2 * _D), jnp.float32),
        pltpu.VMEM((_C, 2 * _D), jnp.float32),
        pltpu.VMEM((_C, 4 * _D), jnp.float32),
        pltpu.SemaphoreType.DMA,
        pltpu.SemaphoreType.DMA,
        pltpu.SemaphoreType.DMA,
        pltpu.SemaphoreType.DMA,
    ],
)(_sc_body)


def _mlp_body(x_ref, wq, bq, wk1, bk1, wv1, bv1, wk2, bk2, wv2, bv2,
              wf1, bf1, wf2r, bf2, wf3, bf3, wf4r, bf4,
              wfc1, bfc1, wfc2, bfc2, wfc3r, bfc3, out_ref):
    dot = lambda a, w: lax.dot_general(a, w, (((1,), (0,)), ((), ())),
                                       preferred_element_type=jnp.float32)
    x = x_ref[:]
    udf = x[:, 0:_D]
    upos = x[:, _D:2 * _D]
    uneg = x[:, 2 * _D:3 * _D]
    uit = x[:, 3 * _D:4 * _D]
    q = dot(udf, wq[:]) + bq[:]
    k1 = dot(upos, wk1[:]) + bk1[:]
    v1 = dot(upos, wv1[:]) + bv1[:]
    k2 = dot(uneg, wk2[:]) + bk2[:]
    v2 = dot(uneg, wv2[:]) + bv2[:]
    h1 = jnp.maximum(dot(k1 * q, wf1[:]) + bf1[:], 0.0)
    s1 = jax.nn.sigmoid(jnp.sum(h1 * wf2r[:], axis=1, keepdims=True) + bf2[:])
    h2 = jnp.maximum(dot(k2 * q, wf3[:]) + bf3[:], 0.0)
    s2 = jax.nn.sigmoid(jnp.sum(h2 * wf4r[:], axis=1, keepdims=True) + bf4[:])
    users = s1 * v1 + s2 * v2
    a = jnp.maximum(dot(users, wfc1[0:_D, :]) + dot(uit, wfc1[_D:2 * _D, :])
                    + bfc1[:], 0.0)
    a = jnp.maximum(dot(a, wfc2[:]) + bfc2[:], 0.0)
    o = jnp.sum(a * wfc3r[:], axis=1) + bfc3[0, 0]
    out_ref[:] = jax.nn.sigmoid(o)


_BM = 512


def _mlp(x, weights):
    n_blocks = _B // _BM
    full = lambda shp: pl.BlockSpec(shp, lambda i: (0,) * len(shp))
    in_specs = [pl.BlockSpec((_BM, 4 * _D), lambda i: (i, 0))]
    in_specs += [full(w.shape) for w in weights]
    return pl.pallas_call(
        _mlp_body,
        grid=(n_blocks,),
        in_specs=in_specs,
        out_specs=pl.BlockSpec((_BM,), lambda i: (i,)),
        out_shape=jax.ShapeDtypeStruct((_B,), jnp.float32),
    )(x, *weights)


def kernel(u, i, adj_G1_index, adj_G1_values, adj_G2_index, adj_G2_values,
           weights_G2, adj_G3_index, adj_G3_values, weights_G3, params):
    p = params
    i32 = jnp.int32
    f32 = jnp.float32

    cat = lambda v, t: jnp.concatenate(
        [v.astype(i32), t.astype(i32)[:, None]], axis=1)
    r_df = cat(adj_G1_values, adj_G1_index)
    r_pos = cat(adj_G2_values, adj_G2_index)
    r_neg = cat(adj_G3_values, adj_G3_index)
    r_it = i.astype(i32)
    pair = lambda r: (r >> 1).reshape(-1)
    hbit = lambda r: (r & 1).astype(f32)
    aux = jnp.concatenate(
        [weights_G2.astype(f32), weights_G3.astype(f32),
         hbit(r_df), hbit(r_pos), hbit(r_neg), hbit(r_it)[:, None],
         jnp.zeros((_B, 3), f32)], axis=1).reshape(-1)

    x = _sc_gather(
        p["users_df"].reshape(-1, 2 * _D), p["users_pos"].reshape(-1, 2 * _D),
        p["users_neg"].reshape(-1, 2 * _D), p["items"].reshape(-1, 2 * _D),
        pair(r_df), pair(r_pos), pair(r_neg), pair(r_it), aux)

    r2 = lambda b: b.reshape(1, -1)
    weights = [
        p["query_W"], r2(p["query_b"]),
        p["key1_W"], r2(p["key1_b"]), p["value1_W"], r2(p["value1_b"]),
        p["key2_W"], r2(p["key2_b"]), p["value2_W"], r2(p["value2_b"]),
        p["f1_W"], r2(p["f1_b"]), p["f2_W"].reshape(1, -1), r2(p["f2_b"]),
        p["f3_W"], r2(p["f3_b"]), p["f4_W"].reshape(1, -1), r2(p["f4_b"]),
        p["fc1_W"], r2(p["fc1_b"]), p["fc2_W"], r2(p["fc2_b"]),
        p["fc3_W"].reshape(1, -1), r2(p["fc3_b"]),
    ]
    return _mlp(x, weights)


# speedup vs baseline: 1.1018x; 1.1018x over previous
import functools

import jax
import jax.numpy as jnp
from jax import lax
from jax.experimental import pallas as pl
from jax.experimental.pallas import tpu as pltpu
from jax.experimental.pallas import tpu_sc as plsc

_B = 16384
_D = 64
_NBR = 5
_K = _NBR + 1
_NC, _NS, _L = 2, 16, 16
_NW = _NC * _NS
_BPW = _B // _NW
_C = 16
_NCH = _BPW // _C
_DJ = _D // _L


def _sc_body(df_t, pos_t, neg_t, it_t,
             idx_df_h, idx_pos_h, idx_neg_h, idx_it_h, w_h,
             out_h,
             idxdf_v, idxpos_v, idxneg_v, idxit_v, w_v,
             rdf_v, rpos_v, rneg_v, rit_v, obuf_v,
             sem_df, sem_pos, sem_neg, sem_it):
    wid = lax.axis_index("s") * _NC + lax.axis_index("c")
    base0 = pl.multiple_of(wid * _BPW, _BPW)

    pltpu.sync_copy(idx_df_h.at[pl.ds(base0 * _K, _BPW * _K)], idxdf_v)
    pltpu.sync_copy(idx_pos_h.at[pl.ds(base0 * _K, _BPW * _K)], idxpos_v)
    pltpu.sync_copy(idx_neg_h.at[pl.ds(base0 * _K, _BPW * _K)], idxneg_v)
    pltpu.sync_copy(idx_it_h.at[pl.ds(base0, _BPW)], idxit_v)
    pltpu.sync_copy(w_h.at[pl.ds(base0 * _L, _BPW * _L)], w_v)

    def chunk(c, carry):
        base = base0 + c * _C
        cp_df = pltpu.async_copy(
            df_t.at[idxdf_v.at[pl.ds(c * _C * _K, _C * _K)]], rdf_v, sem_df)
        cp_pos = pltpu.async_copy(
            pos_t.at[idxpos_v.at[pl.ds(c * _C * _K, _C * _K)]], rpos_v, sem_pos)
        cp_neg = pltpu.async_copy(
            neg_t.at[idxneg_v.at[pl.ds(c * _C * _K, _C * _K)]], rneg_v, sem_neg)
        cp_it = pltpu.async_copy(
            it_t.at[idxit_v.at[pl.ds(c * _C, _C)]], rit_v, sem_it)
        cp_df.wait()
        cp_pos.wait()
        cp_neg.wait()
        cp_it.wait()

        def row(b, carry2):
            r0 = b * _K
            wrow = w_v[pl.ds((c * _C + b) * _L, _L)]
            for j in range(_DJ):
                sl = pl.ds(j * _L, _L)
                e = rdf_v[r0, sl]
                s = e
                q = e * e
                for n in range(1, _NBR):
                    e = rdf_v[r0 + n, sl]
                    s = s + e
                    q = q + e * e
                obuf_v[b, sl] = s * s - q + rdf_v[r0 + _NBR, sl]
                accp = rpos_v[r0 + _NBR, sl]
                accn = rneg_v[r0 + _NBR, sl]
                for n in range(_NBR):
                    accp = accp + wrow[n] * rpos_v[r0 + n, sl]
                    accn = accn + wrow[5 + n] * rneg_v[r0 + n, sl]
                obuf_v[b, pl.ds(_D + j * _L, _L)] = accp
                obuf_v[b, pl.ds(2 * _D + j * _L, _L)] = accn
                obuf_v[b, pl.ds(3 * _D + j * _L, _L)] = rit_v[b, sl]
            return carry2

        lax.fori_loop(0, _C, row, 0)
        pltpu.sync_copy(obuf_v, out_h.at[pl.ds(base, _C)])
        return carry

    lax.fori_loop(0, _NCH, chunk, 0)


_sc_gather = functools.partial(
    pl.kernel,
    out_type=jax.ShapeDtypeStruct((_B, 4 * _D), jnp.float32),
    mesh=plsc.VectorSubcoreMesh(core_axis_name="c", subcore_axis_name="s",
                                num_cores=_NC, num_subcores=_NS),
    compiler_params=pltpu.CompilerParams(use_tc_tiling_on_sc=False),
    scratch_types=[
        pltpu.VMEM((_BPW * _K,), jnp.int32),
        pltpu.VMEM((_BPW * _K,), jnp.int32),
        pltpu.VMEM((_BPW * _K,), jnp.int32),
        pltpu.VMEM((_BPW,), jnp.int32),
        pltpu.VMEM((_BPW * _L,), jnp.float32),
        pltpu.VMEM((_C * _K, _D), jnp.float32),
        pltpu.VMEM((_C * _K, _D), jnp.float32),
        pltpu.VMEM((_C * _K, _D), jnp.float32),
        pltpu.VMEM((_C, _D), jnp.float32),
        pltpu.VMEM((_C, 4 * _D), jnp.float32),
        pltpu.SemaphoreType.DMA,
        pltpu.SemaphoreType.DMA,
        pltpu.SemaphoreType.DMA,
        pltpu.SemaphoreType.DMA,
    ],
)(_sc_body)


def _mlp_body(x_ref, wq, bq, wk1, bk1, wv1, bv1, wk2, bk2, wv2, bv2,
              wf1, bf1, wf2r, bf2, wf3, bf3, wf4r, bf4,
              wfc1, bfc1, wfc2, bfc2, wfc3r, bfc3, out_ref):
    bf = jnp.bfloat16
    dot = lambda a, w: lax.dot_general(a.astype(bf), w, (((1,), (0,)), ((), ())),
                                       preferred_element_type=jnp.float32)
    x = x_ref[:]
    udf = x[:, 0:_D]
    upos = x[:, _D:2 * _D]
    uneg = x[:, 2 * _D:3 * _D]
    uit = x[:, 3 * _D:4 * _D]
    q = dot(udf, wq[:]) + bq[:]
    k1 = dot(upos, wk1[:]) + bk1[:]
    v1 = dot(upos, wv1[:]) + bv1[:]
    k2 = dot(uneg, wk2[:]) + bk2[:]
    v2 = dot(uneg, wv2[:]) + bv2[:]
    h1 = jnp.maximum(dot(k1 * q, wf1[:]) + bf1[:], 0.0)
    s1 = jax.nn.sigmoid(jnp.sum(h1 * wf2r[:], axis=1, keepdims=True) + bf2[:])
    h2 = jnp.maximum(dot(k2 * q, wf3[:]) + bf3[:], 0.0)
    s2 = jax.nn.sigmoid(jnp.sum(h2 * wf4r[:], axis=1, keepdims=True) + bf4[:])
    users = s1 * v1 + s2 * v2
    a = jnp.maximum(dot(users, wfc1[0:_D, :]) + dot(uit, wfc1[_D:2 * _D, :])
                    + bfc1[:], 0.0)
    a = jnp.maximum(dot(a, wfc2[:]) + bfc2[:], 0.0)
    o = jnp.sum(a * wfc3r[:], axis=1) + bfc3[0, 0]
    out_ref[:] = jax.nn.sigmoid(o)


_BM = 512


def _mlp(x, weights):
    n_blocks = _B // _BM
    full = lambda shp: pl.BlockSpec(shp, lambda i: (0,) * len(shp))
    in_specs = [pl.BlockSpec((_BM, 4 * _D), lambda i: (i, 0))]
    in_specs += [full(w.shape) for w in weights]
    return pl.pallas_call(
        _mlp_body,
        grid=(n_blocks,),
        in_specs=in_specs,
        out_specs=pl.BlockSpec((_BM,), lambda i: (i,)),
        out_shape=jax.ShapeDtypeStruct((_B,), jnp.float32),
    )(x, *weights)


def kernel(u, i, adj_G1_index, adj_G1_values, adj_G2_index, adj_G2_values,
           weights_G2, adj_G3_index, adj_G3_values, weights_G3, params):
    p = params
    i32 = jnp.int32
    f32 = jnp.float32
    bf = jnp.bfloat16

    pack = lambda v, t: jnp.concatenate(
        [v.astype(i32), t.astype(i32)[:, None]], axis=1).reshape(-1)
    idx_df = pack(adj_G1_values, adj_G1_index)
    idx_pos = pack(adj_G2_values, adj_G2_index)
    idx_neg = pack(adj_G3_values, adj_G3_index)
    idx_it = i.astype(i32)
    w = jnp.concatenate(
        [weights_G2.astype(f32), weights_G3.astype(f32),
         jnp.zeros((_B, _L - 2 * _NBR), f32)], axis=1).reshape(-1)

    x = _sc_gather(p["users_df"], p["users_pos"], p["users_neg"], p["items"],
                   idx_df, idx_pos, idx_neg, idx_it, w)

    r2 = lambda b: b.reshape(1, -1)
    wb = lambda m: m.astype(bf)
    weights = [
        wb(p["query_W"]), r2(p["query_b"]),
        wb(p["key1_W"]), r2(p["key1_b"]), wb(p["value1_W"]), r2(p["value1_b"]),
        wb(p["key2_W"]), r2(p["key2_b"]), wb(p["value2_W"]), r2(p["value2_b"]),
        wb(p["f1_W"]), r2(p["f1_b"]), p["f2_W"].reshape(1, -1), r2(p["f2_b"]),
        wb(p["f3_W"]), r2(p["f3_b"]), p["f4_W"].reshape(1, -1), r2(p["f4_b"]),
        wb(p["fc1_W"]), r2(p["fc1_b"]), wb(p["fc2_W"]), r2(p["fc2_b"]),
        p["fc3_W"].reshape(1, -1), r2(p["fc3_b"]),
    ]
    return _mlp(x, weights)
